# baseline (speedup 1.0000x reference)
import functools

import jax
import jax.numpy as jnp
from jax import lax
from jax.experimental import pallas as pl

N_RANKS = (10000, 160000, 80000, 20000)
C = 128
OUT = 64
L = 2


def _mm_kernel(x_ref, w_ref, o_ref):
    o_ref[...] = jnp.dot(x_ref[...], w_ref[...],
                         preferred_element_type=jnp.float32)


def _mm(x, w, bm=1024):
    n, k = x.shape
    m = w.shape[1]
    return pl.pallas_call(
        _mm_kernel,
        grid=(pl.cdiv(n, bm),),
        in_specs=[pl.BlockSpec((bm, k), lambda i: (i, 0)),
                  pl.BlockSpec((k, m), lambda i: (0, 0))],
        out_specs=pl.BlockSpec((bm, m), lambda i: (i, 0)),
        out_shape=jax.ShapeDtypeStruct((n, m), jnp.float32),
    )(x, w)


def _head_kernel(x_ref, w_ref, b_ref, o_ref):
    logits = jnp.dot(x_ref[...], w_ref[...],
                     preferred_element_type=jnp.float32) + b_ref[...]
    z = logits - jnp.max(logits, axis=1, keepdims=True)
    e = jnp.exp(z)
    o_ref[...] = e / jnp.sum(e, axis=1, keepdims=True)


def _head(x, w, b, bm=1024):
    n, k = x.shape
    m = w.shape[1]
    b2 = b.reshape(1, m)
    return pl.pallas_call(
        _head_kernel,
        grid=(pl.cdiv(n, bm),),
        in_specs=[pl.BlockSpec((bm, k), lambda i: (i, 0)),
                  pl.BlockSpec((k, m), lambda i: (0, 0)),
                  pl.BlockSpec((1, m), lambda i: (0, 0))],
        out_specs=pl.BlockSpec((bm, m), lambda i: (i, 0)),
        out_shape=jax.ShapeDtypeStruct((n, m), jnp.float32),
    )(x, w, b2)


def kernel(f0, f1, f2, f3, adj0, adj1, adj2, adj3, inc1, inc2, inc3,
           W_same, W_low, W_up, W_lin, b_lin):
    feats = [f0, f1, f2, f3]
    adjs = [adj0.astype(jnp.int32), adj1.astype(jnp.int32),
            adj2.astype(jnp.int32), adj3.astype(jnp.int32)]
    incs = [None, inc1.astype(jnp.int32), inc2.astype(jnp.int32),
            inc3.astype(jnp.int32)]

    widths = (2, 3, 3, 2)
    bases = []
    acc = 0
    for s in range(4):
        bases.append(acc)
        acc += N_RANKS[s] * widths[s]
    same_piece = (0, 0, 0, 0)
    low_piece = (1, 1, 1, None)
    up_piece = (None, 2, 2, 1)

    srcs, dsts = [], []
    for r in range(4):
        s_parts = [bases[r] + adjs[r][1] * widths[r] + same_piece[r]]
        d_parts = [adjs[r][0]]
        if r > 0:
            s_parts.append(bases[r - 1] + incs[r][0] * widths[r - 1]
                           + low_piece[r - 1])
            d_parts.append(incs[r][1])
        if r < 3:
            s_parts.append(bases[r + 1] + incs[r + 1][1] * widths[r + 1]
                           + up_piece[r + 1])
            d_parts.append(incs[r + 1][0])
        srcs.append(jnp.concatenate(s_parts))
        dsts.append(jnp.concatenate(d_parts))

    for l in range(L):
        wcats = []
        for s in range(4):
            parts = [W_same[l, s]]
            if s < 3:
                parts.append(W_low[l, s])
            if s > 0:
                parts.append(W_up[l, s - 1])
            wcats.append(jnp.concatenate(parts, axis=1))
        outs = [_mm(feats[s], wcats[s]) for s in range(4)]
        table = jnp.concatenate([o.reshape(-1, C) for o in outs], axis=0)
        feats = [jax.nn.sigmoid(
                    jax.ops.segment_sum(table[srcs[r]], dsts[r],
                                        num_segments=N_RANKS[r]))
                 for r in range(4)]

    return _head(feats[0], W_lin, b_lin.astype(jnp.float32))


# speedup vs baseline: 1.9951x; 1.9951x over previous
import functools

import jax
import jax.numpy as jnp
from jax import lax
from jax.experimental import pallas as pl
from jax.experimental.pallas import tpu as pltpu
from jax.experimental.pallas import tpu_sc as plsc

C = 128
N0, N1, N2 = 10000, 160000, 80000
NP0, NP1 = 10240, 163840
WIN = 1024
CAP, KB, BW = 256, 2, 128
NSUB = 16


def _sigmoid(x):
    return 1.0 / (1.0 + jnp.exp(-x))


def _mm_kernel(sig, x_ref, w_ref, o_ref):
    x = x_ref[...]
    if sig:
        x = _sigmoid(x)
    o_ref[...] = jnp.dot(x, w_ref[...], preferred_element_type=jnp.float32)


def _mm(x, w, sig=False, bm=1024):
    n, k = x.shape
    m = w.shape[1]
    return pl.pallas_call(
        functools.partial(_mm_kernel, sig),
        grid=(pl.cdiv(n, bm),),
        in_specs=[pl.BlockSpec((bm, k), lambda i: (i, 0)),
                  pl.BlockSpec((k, m), lambda i: (0, 0))],
        out_specs=pl.BlockSpec((bm, m), lambda i: (i, 0)),
        out_shape=jax.ShapeDtypeStruct((n, m), jnp.float32),
    )(x, w)


def _head_kernel(x_ref, w_ref, b_ref, o_ref):
    x = _sigmoid(x_ref[...])
    logits = jnp.dot(x, w_ref[...], preferred_element_type=jnp.float32)
    logits = logits + b_ref[...]
    z = logits - jnp.max(logits, axis=1, keepdims=True)
    e = jnp.exp(z)
    o_ref[...] = e / jnp.sum(e, axis=1, keepdims=True)


def _head(x, w, b, n_out, bm=1024):
    k = x.shape[1]
    m = w.shape[1]
    return pl.pallas_call(
        _head_kernel,
        grid=(pl.cdiv(n_out, bm),),
        in_specs=[pl.BlockSpec((bm, k), lambda i: (i, 0)),
                  pl.BlockSpec((k, m), lambda i: (0, 0)),
                  pl.BlockSpec((1, m), lambda i: (0, 0))],
        out_specs=pl.BlockSpec((bm, m), lambda i: (i, 0)),
        out_shape=jax.ShapeDtypeStruct((n_out, m), jnp.float32),
    )(x, w, b.reshape(1, m))


def _make_segsum(np_rows, chsize, nch, ep):
    assert np_rows == nch * chsize and nch % 2 == 0
    assert chsize % (NSUB * 64) == 0 and ep % (NSUB * WIN) == 0
    ts = chsize // NSUB
    share = ep // NSUB
    nwin = share // WIN
    mesh = plsc.VectorSubcoreMesh(core_axis_name="c", subcore_axis_name="s")

    @functools.partial(
        pl.kernel, mesh=mesh,
        out_type=jax.ShapeDtypeStruct((np_rows, C), jnp.float32),
        scratch_types=[
            pltpu.VMEM_SHARED((chsize, C), jnp.float32),
            pltpu.VMEM((WIN,), jnp.int32),
            pltpu.VMEM((WIN,), jnp.int32),
            pltpu.VMEM((KB, BW), jnp.int32),
            pltpu.VMEM((KB, BW), jnp.int32),
            pltpu.VMEM((CAP, C), jnp.float32),
            pltpu.VMEM((32, C), jnp.float32),
            pltpu.SemaphoreType.DMA,
        ],
        compiler_params=pltpu.CompilerParams(needs_layout_passes=False),
    )
    def k(table, src, dst, out, acc, dwin, swin, gidx, ldst, rows, zbuf, gsem):
        cid = lax.axis_index("c")
        sid = lax.axis_index("s")
        zvec = jnp.zeros((16,), jnp.float32)
        zivec = jnp.zeros((16,), jnp.int32)

        def _zb(i, _):
            zbuf[i // 8, pl.ds((i % 8) * 16, 16)] = zvec
            return 0
        lax.fori_loop(0, 256, _zb, 0)
        for j in range(KB):
            for q in range(BW // 16):
                gidx[j, pl.ds(q * 16, 16)] = zivec
                ldst[j, pl.ds(q * 16, 16)] = zivec

        def _flush(cnt):
            cps = [pltpu.async_copy(table.at[gidx.at[j]],
                                    rows.at[pl.ds(j * BW, BW)], gsem)
                   for j in range(KB)]
            for cp in cps:
                cp.wait()

            def _zr(i, _):
                for q in range(C // 16):
                    rows[i, pl.ds(q * 16, 16)] = zvec
                return 0
            lax.fori_loop(cnt, CAP, _zr, 0)
            for j in range(KB):
                pltpu.sync_copy(rows.at[pl.ds(j * BW, BW)],
                                acc.at[ldst.at[j]], add=True)
            return jnp.int32(0)

        for kk in range(nch // 2):
            lo = (2 * kk + cid) * chsize
            hi = lo + chsize
            for p in range(ts // 32):
                pltpu.sync_copy(zbuf, acc.at[pl.ds(sid * ts + p * 32, 32)])
            plsc.subcore_barrier()

            def _win(w, cnt):
                base = sid * share + w * WIN
                pltpu.sync_copy(dst.at[pl.ds(base, WIN)], dwin)
                pltpu.sync_copy(src.at[pl.ds(base, WIN)], swin)

                def _vec(i, cnt):
                    d = dwin[pl.ds(i * 16, 16)]
                    s = swin[pl.ds(i * 16, 16)]
                    m = (d >= lo) & (d < hi)
                    mi = jnp.where(m, jnp.full((16,), 1, jnp.int32),
                                   jnp.full((16,), 0, jnp.int32))
                    pos = cnt + plsc.cumsum(mi) - 1
                    pj, pq = pos // BW, pos % BW
                    plsc.store_scatter(gidx, [pj, pq], s, mask=m)
                    plsc.store_scatter(ldst, [pj, pq], d - lo, mask=m)
                    cnt = cnt + jnp.sum(mi)
                    return lax.cond(cnt > CAP - 16, _flush,
                                    lambda c: c, cnt)
                return lax.fori_loop(0, WIN // 16, _vec, cnt)

            cnt = lax.fori_loop(0, nwin, _win, jnp.int32(0))
            _flush(cnt)
            plsc.subcore_barrier()
            for p in range(ts // 64):
                off = sid * ts + p * 64
                pltpu.sync_copy(acc.at[pl.ds(off, 64)],
                                out.at[pl.ds(lo + off, 64)])

    return k


_segsum_r0 = _make_segsum(NP0, 5120, 2, 491520)
_segsum_r1 = _make_segsum(NP1, 10240, 16, 884736)


def _pad_edges(src, dst, ep):
    e = src.shape[0]
    return (jnp.pad(src, (0, ep - e)),
            jnp.pad(dst, (0, ep - e), constant_values=-1))


def kernel(f0, f1, f2, f3, adj0, adj1, adj2, adj3, inc1, inc2, inc3,
           W_same, W_low, W_up, W_lin, b_lin):
    a0 = adj0.astype(jnp.int32)
    a1 = adj1.astype(jnp.int32)
    i1 = inc1.astype(jnp.int32)
    i2 = inc2.astype(jnp.int32)

    bB = 2 * N0
    bC = bB + 2 * N1
    src10 = jnp.concatenate([2 * a0[1], bB + 2 * i1[1] + 1])
    dst10 = jnp.concatenate([a0[0], i1[0]])
    src11 = jnp.concatenate([bB + 2 * a1[1], 2 * i1[0] + 1, bC + i2[1]])
    dst11 = jnp.concatenate([a1[0], i1[1], i2[0]])
    src10, dst10 = _pad_edges(src10, dst10, 491520)
    src11, dst11 = _pad_edges(src11, dst11, 884736)

    wA = jnp.concatenate([W_same[0, 0], W_low[0, 0]], axis=1)
    wB = jnp.concatenate([W_same[0, 1], W_up[0, 0]], axis=1)
    tab1 = jnp.concatenate([
        _mm(f0, wA).reshape(-1, C),
        _mm(f1, wB).reshape(-1, C),
        _mm(f2, W_up[0, 1]),
    ], axis=0)

    m0 = _segsum_r0(tab1, src10, dst10)
    m1 = _segsum_r1(tab1, src11, dst11)

    src20 = jnp.concatenate([a0[1], NP0 + i1[1]])
    dst20 = jnp.concatenate([a0[0], i1[0]])
    src20, dst20 = _pad_edges(src20, dst20, 491520)

    tab2 = jnp.concatenate([
        _mm(m0, W_same[1, 0], sig=True),
        _mm(m1, W_up[1, 0], sig=True),
    ], axis=0)

    m0f = _segsum_r0(tab2, src20, dst20)

    return _head(m0f, W_lin, b_lin.astype(jnp.float32), N0)


# speedup vs baseline: 4.1514x; 2.0808x over previous
import functools

import jax
import jax.numpy as jnp
from jax import lax
from jax.experimental import pallas as pl
from jax.experimental.pallas import tpu as pltpu
from jax.experimental.pallas import tpu_sc as plsc

C = 128
N0, N1, N2 = 10000, 160000, 80000
NP0, NP1 = 10240, 163840
WIN = 1024
BW = 128
NSUB = 16


def _sigmoid(x):
    return 1.0 / (1.0 + jnp.exp(-x))


def _mm_kernel(sig, x_ref, w_ref, o_ref):
    x = x_ref[...]
    if sig:
        x = _sigmoid(x)
    o_ref[...] = jnp.dot(x, w_ref[...], preferred_element_type=jnp.float32)


def _mm(x, w, sig=False, bm=1024):
    n, k = x.shape
    m = w.shape[1]
    return pl.pallas_call(
        functools.partial(_mm_kernel, sig),
        grid=(pl.cdiv(n, bm),),
        in_specs=[pl.BlockSpec((bm, k), lambda i: (i, 0)),
                  pl.BlockSpec((k, m), lambda i: (0, 0))],
        out_specs=pl.BlockSpec((bm, m), lambda i: (i, 0)),
        out_shape=jax.ShapeDtypeStruct((n, m), jnp.float32),
    )(x, w)


def _head_kernel(x_ref, w_ref, b_ref, o_ref):
    x = _sigmoid(x_ref[...])
    logits = jnp.dot(x, w_ref[...], preferred_element_type=jnp.float32)
    logits = logits + b_ref[...]
    z = logits - jnp.max(logits, axis=1, keepdims=True)
    e = jnp.exp(z)
    o_ref[...] = e / jnp.sum(e, axis=1, keepdims=True)


def _head(x, w, b, n_out, bm=1024):
    k = x.shape[1]
    m = w.shape[1]
    return pl.pallas_call(
        _head_kernel,
        grid=(pl.cdiv(n_out, bm),),
        in_specs=[pl.BlockSpec((bm, k), lambda i: (i, 0)),
                  pl.BlockSpec((k, m), lambda i: (0, 0)),
                  pl.BlockSpec((1, m), lambda i: (0, 0))],
        out_specs=pl.BlockSpec((bm, m), lambda i: (i, 0)),
        out_shape=jax.ShapeDtypeStruct((n_out, m), jnp.float32),
    )(x, w, b.reshape(1, m))


def _make_segsum(np_rows, chsize, nch, ep):
    assert np_rows == nch * chsize and nch % 2 == 0
    assert chsize % (NSUB * 64) == 0 and ep % (NSUB * WIN) == 0
    ts = chsize // NSUB
    share = ep // NSUB
    nwin = share // WIN
    assert nwin % 2 == 0
    mesh = plsc.VectorSubcoreMesh(core_axis_name="c", subcore_axis_name="s")

    @functools.partial(
        pl.kernel, mesh=mesh,
        out_type=jax.ShapeDtypeStruct((np_rows, C), jnp.float32),
        scratch_types=[
            pltpu.VMEM_SHARED((chsize, C), jnp.float32),
            pltpu.VMEM((2, 2, WIN), jnp.int32),
            pltpu.VMEM((4, BW), jnp.int32),
            pltpu.VMEM((4, BW), jnp.int32),
            pltpu.VMEM((2 * BW, C), jnp.float32),
            pltpu.VMEM((32, C), jnp.float32),
            pltpu.SemaphoreType.DMA,
            pltpu.SemaphoreType.DMA,
            pltpu.SemaphoreType.DMA,
        ],
        compiler_params=pltpu.CompilerParams(needs_layout_passes=False),
    )
    def k(table, edges, out, acc, ewin, gidx, ldst, rows, zbuf, wsem, gsem,
          ssem):
        cid = lax.axis_index("c")
        sid = lax.axis_index("s")
        zvec = jnp.zeros((16,), jnp.float32)
        zivec = jnp.zeros((16,), jnp.int32)

        def _wait_w():
            pltpu.make_async_copy(edges.at[0], ewin.at[0], wsem).wait()

        def _wait_g():
            pltpu.make_async_copy(table.at[gidx.at[0]],
                                  rows.at[pl.ds(0, BW)], gsem).wait()

        def _wait_s(*_):
            pltpu.make_async_copy(rows.at[pl.ds(0, BW)],
                                  acc.at[ldst.at[0]], ssem).wait()
            return 0

        def _fire_g(f):
            pltpu.async_copy(table.at[gidx.at[f & 3]],
                             rows.at[pl.ds((f & 1) * BW, BW)], gsem)

        def _fire_s(f):
            pltpu.async_copy(rows.at[pl.ds((f & 1) * BW, BW)],
                             acc.at[ldst.at[f & 3]], ssem, add=True)

        def _zb(i, _):
            zbuf[i // 8, pl.ds((i % 8) * 16, 16)] = zvec
            return 0
        lax.fori_loop(0, 256, _zb, 0)
        for j in range(4):
            for q in range(BW // 16):
                gidx[j, pl.ds(q * 16, 16)] = zivec
                ldst[j, pl.ds(q * 16, 16)] = zivec

        for kk in range(nch // 2):
            lo = (2 * kk + cid) * chsize
            hi = lo + chsize
            for p in range(ts // 32):
                pltpu.sync_copy(zbuf, acc.at[pl.ds(sid * ts + p * 32, 32)])
            plsc.subcore_barrier()

            wbase = sid * nwin
            pltpu.async_copy(edges.at[wbase], ewin.at[0], wsem)

            def _fire(c):
                f = (c >> 7) - 1
                lax.cond(f >= 2, _wait_s, lambda _: 0, 0)

                def _gs(_):
                    _wait_g()
                    _fire_s(f - 1)
                    return 0
                lax.cond(f >= 1, _gs, lambda _: 0, 0)
                _fire_g(f)
                return c

            def _scan(w, par, cnt, last):
                _wait_w()
                if not last:
                    pltpu.async_copy(edges.at[wbase + w + 1],
                                     ewin.at[1 - par], wsem)

                def _vec(i, cnt):
                    d = ewin[par, 0, pl.ds(i * 16, 16)]
                    s = ewin[par, 1, pl.ds(i * 16, 16)]
                    m = (d >= lo) & (d < hi)
                    mi = jnp.where(m, jnp.full((16,), 1, jnp.int32),
                                   jnp.full((16,), 0, jnp.int32))
                    pos = cnt + plsc.cumsum(mi) - 1
                    pj = (pos >> 7) & 3
                    pq = pos & (BW - 1)
                    plsc.store_scatter(gidx, [pj, pq], s, mask=m)
                    plsc.store_scatter(ldst, [pj, pq], d - lo, mask=m)
                    newc = cnt + jnp.sum(mi)
                    return lax.cond((newc >> 7) > (cnt >> 7), _fire,
                                    lambda c: c, newc)
                return lax.fori_loop(0, WIN // 16, _vec, cnt)

            def _win2b(w2, cnt):
                w = 2 * w2
                cnt = _scan(w, 0, cnt, False)
                return _scan(w + 1, 1, cnt, False)

            cnt = lax.fori_loop(0, nwin // 2 - 1, _win2b, jnp.int32(0))
            cnt = _scan(nwin - 2, 0, cnt, False)
            cnt = _scan(nwin - 1, 1, cnt, True)

            f_t = cnt >> 7
            rem = cnt & (BW - 1)
            p_t = f_t & 1
            lax.cond(f_t >= 2, _wait_s, lambda _: 0, 0)

            def _drain_full(_):
                _wait_g()
                _fire_s(f_t - 1)
                return 0
            lax.cond(f_t >= 1, _drain_full, lambda _: 0, 0)

            def _drain_part(_):
                _fire_g(f_t)
                _wait_g()

                def _zr(i, _):
                    for q in range(C // 16):
                        rows[i, pl.ds(q * 16, 16)] = zvec
                    return 0
                lax.fori_loop(p_t * BW + rem, p_t * BW + BW, _zr, 0)
                _fire_s(f_t)
                return 0
            lax.cond(rem > 0, _drain_part, lambda _: 0, 0)

            n_out = (jnp.where(f_t > 0, 1, 0).astype(jnp.int32)
                     + jnp.where(rem > 0, 1, 0).astype(jnp.int32))
            lax.fori_loop(0, n_out, _wait_s, 0)

            plsc.subcore_barrier()
            for p in range(ts // 64):
                off = sid * ts + p * 64
                pltpu.sync_copy(acc.at[pl.ds(off, 64)],
                                out.at[pl.ds(lo + off, 64)])

    return k


_segsum_r0 = _make_segsum(NP0, 5120, 2, 491520)
_segsum_r1 = _make_segsum(NP1, 10240, 16, 884736)


def _pad_edges(src, dst, ep):
    e = src.shape[0]
    src = jnp.pad(src, (0, ep - e))
    dst = jnp.pad(dst, (0, ep - e), constant_values=-1)
    return jnp.stack([dst.reshape(-1, WIN), src.reshape(-1, WIN)], axis=1)


def kernel(f0, f1, f2, f3, adj0, adj1, adj2, adj3, inc1, inc2, inc3,
           W_same, W_low, W_up, W_lin, b_lin):
    a0 = adj0.astype(jnp.int32)
    a1 = adj1.astype(jnp.int32)
    i1 = inc1.astype(jnp.int32)
    i2 = inc2.astype(jnp.int32)

    bB = 2 * N0
    bC = bB + 2 * N1
    src10 = jnp.concatenate([2 * a0[1], bB + 2 * i1[1] + 1])
    dst10 = jnp.concatenate([a0[0], i1[0]])
    src11 = jnp.concatenate([bB + 2 * a1[1], 2 * i1[0] + 1, bC + i2[1]])
    dst11 = jnp.concatenate([a1[0], i1[1], i2[0]])
    e10 = _pad_edges(src10, dst10, 491520)
    e11 = _pad_edges(src11, dst11, 884736)

    wA = jnp.concatenate([W_same[0, 0], W_low[0, 0]], axis=1)
    wB = jnp.concatenate([W_same[0, 1], W_up[0, 0]], axis=1)
    tab1 = jnp.concatenate([
        _mm(f0, wA).reshape(-1, C),
        _mm(f1, wB).reshape(-1, C),
        _mm(f2, W_up[0, 1]),
    ], axis=0)

    m0 = _segsum_r0(tab1, e10)
    m1 = _segsum_r1(tab1, e11)

    src20 = jnp.concatenate([a0[1], NP0 + i1[1]])
    dst20 = jnp.concatenate([a0[0], i1[0]])
    e20 = _pad_edges(src20, dst20, 491520)

    tab2 = jnp.concatenate([
        _mm(m0, W_same[1, 0], sig=True),
        _mm(m1, W_up[1, 0], sig=True),
    ], axis=0)

    m0f = _segsum_r0(tab2, e20)

    return _head(m0f, W_lin, b_lin.astype(jnp.float32), N0)


# speedup vs baseline: 4.2050x; 1.0129x over previous
import functools

import jax
import jax.numpy as jnp
from jax import lax
from jax.experimental import pallas as pl
from jax.experimental.pallas import tpu as pltpu
from jax.experimental.pallas import tpu_sc as plsc

C = 128
N0, N1, N2 = 10000, 160000, 80000
NP0, NP1 = 10240, 172032
NSUB = 16


def _sigmoid(x):
    return 1.0 / (1.0 + jnp.exp(-x))


def _mm_kernel(sig, x_ref, w_ref, o_ref):
    x = x_ref[...]
    if sig:
        x = _sigmoid(x)
    o_ref[...] = jnp.dot(x, w_ref[...], preferred_element_type=jnp.float32)


def _mm(x, w, sig=False, bm=1024):
    n, k = x.shape
    m = w.shape[1]
    return pl.pallas_call(
        functools.partial(_mm_kernel, sig),
        grid=(pl.cdiv(n, bm),),
        in_specs=[pl.BlockSpec((bm, k), lambda i: (i, 0)),
                  pl.BlockSpec((k, m), lambda i: (0, 0))],
        out_specs=pl.BlockSpec((bm, m), lambda i: (i, 0)),
        out_shape=jax.ShapeDtypeStruct((n, m), jnp.float32),
    )(x, w)


def _head_kernel(x_ref, w_ref, b_ref, o_ref):
    x = _sigmoid(x_ref[...])
    logits = jnp.dot(x, w_ref[...], preferred_element_type=jnp.float32)
    logits = logits + b_ref[...]
    z = logits - jnp.max(logits, axis=1, keepdims=True)
    e = jnp.exp(z)
    o_ref[...] = e / jnp.sum(e, axis=1, keepdims=True)


def _head(x, w, b, n_out, bm=1024):
    k = x.shape[1]
    m = w.shape[1]
    return pl.pallas_call(
        _head_kernel,
        grid=(pl.cdiv(n_out, bm),),
        in_specs=[pl.BlockSpec((bm, k), lambda i: (i, 0)),
                  pl.BlockSpec((k, m), lambda i: (0, 0)),
                  pl.BlockSpec((1, m), lambda i: (0, 0))],
        out_specs=pl.BlockSpec((bm, m), lambda i: (i, 0)),
        out_shape=jax.ShapeDtypeStruct((n_out, m), jnp.float32),
    )(x, w, b.reshape(1, m))


def _make_segsum(np_rows, chsize, nch, ep, bw, win):
    assert np_rows == nch * chsize and nch % 2 == 0
    assert chsize % (NSUB * 64) == 0 and ep % (NSUB * win) == 0
    assert bw >= 32 and bw % 16 == 0 and (bw & (bw - 1)) == 0
    bsh = bw.bit_length() - 1
    ts = chsize // NSUB
    share = ep // NSUB
    nwin = share // win
    assert nwin % 2 == 0
    mesh = plsc.VectorSubcoreMesh(core_axis_name="c", subcore_axis_name="s")

    @functools.partial(
        pl.kernel, mesh=mesh,
        out_type=jax.ShapeDtypeStruct((np_rows, C), jnp.float32),
        scratch_types=[
            pltpu.VMEM_SHARED((chsize, C), jnp.float32),
            pltpu.VMEM((2, 2, win), jnp.int32),
            pltpu.VMEM((4, bw), jnp.int32),
            pltpu.VMEM((4, bw), jnp.int32),
            pltpu.VMEM((2 * bw, C), jnp.float32),
            pltpu.VMEM((32, C), jnp.float32),
            pltpu.SemaphoreType.DMA,
            pltpu.SemaphoreType.DMA,
            pltpu.SemaphoreType.DMA,
        ],
        compiler_params=pltpu.CompilerParams(needs_layout_passes=False),
    )
    def k(table, edges, out, acc, ewin, gidx, ldst, rows, zbuf, wsem, gsem,
          ssem):
        cid = lax.axis_index("c")
        sid = lax.axis_index("s")
        zvec = jnp.zeros((16,), jnp.float32)
        zivec = jnp.zeros((16,), jnp.int32)

        def _wait_w():
            pltpu.make_async_copy(edges.at[0], ewin.at[0], wsem).wait()

        def _wait_g():
            pltpu.make_async_copy(table.at[gidx.at[0]],
                                  rows.at[pl.ds(0, bw)], gsem).wait()

        def _wait_s(*_):
            pltpu.make_async_copy(rows.at[pl.ds(0, bw)],
                                  acc.at[ldst.at[0]], ssem).wait()
            return 0

        def _fire_g(f):
            pltpu.async_copy(table.at[gidx.at[f & 3]],
                             rows.at[pl.ds((f & 1) * bw, bw)], gsem)

        def _fire_s(f):
            pltpu.async_copy(rows.at[pl.ds((f & 1) * bw, bw)],
                             acc.at[ldst.at[f & 3]], ssem, add=True)

        def _zb(i, _):
            zbuf[i // 8, pl.ds((i % 8) * 16, 16)] = zvec
            return 0
        lax.fori_loop(0, 256, _zb, 0)
        for j in range(4):
            for q in range(bw // 16):
                gidx[j, pl.ds(q * 16, 16)] = zivec
                ldst[j, pl.ds(q * 16, 16)] = zivec

        for kk in range(nch // 2):
            lo = (2 * kk + cid) * chsize
            hi = lo + chsize
            for p in range(ts // 32):
                pltpu.sync_copy(zbuf, acc.at[pl.ds(sid * ts + p * 32, 32)])
            plsc.subcore_barrier()

            wbase = sid * nwin
            pltpu.async_copy(edges.at[wbase], ewin.at[0], wsem)

            def _fire(c):
                f = (c >> bsh) - 1
                lax.cond(f >= 2, _wait_s, lambda _: 0, 0)

                def _gs(_):
                    _wait_g()
                    _fire_s(f - 1)
                    return 0
                lax.cond(f >= 1, _gs, lambda _: 0, 0)
                _fire_g(f)
                return c

            def _scan(w, par, cnt, last):
                _wait_w()
                if not last:
                    pltpu.async_copy(edges.at[wbase + w + 1],
                                     ewin.at[1 - par], wsem)

                def _vec(i, cnt):
                    d = ewin[par, 0, pl.ds(i * 16, 16)]
                    s = ewin[par, 1, pl.ds(i * 16, 16)]
                    m = (d >= lo) & (d < hi)
                    mi = jnp.where(m, jnp.full((16,), 1, jnp.int32),
                                   jnp.full((16,), 0, jnp.int32))
                    pos = cnt + plsc.cumsum(mi) - 1
                    pj = (pos >> bsh) & 3
                    pq = pos & (bw - 1)
                    plsc.store_scatter(gidx, [pj, pq], s, mask=m)
                    plsc.store_scatter(ldst, [pj, pq], d - lo, mask=m)
                    newc = cnt + jnp.sum(mi)
                    return lax.cond((newc >> bsh) > (cnt >> bsh), _fire,
                                    lambda c: c, newc)
                return lax.fori_loop(0, win // 16, _vec, cnt)

            def _win2b(w2, cnt):
                w = 2 * w2
                cnt = _scan(w, 0, cnt, False)
                return _scan(w + 1, 1, cnt, False)

            cnt = lax.fori_loop(0, nwin // 2 - 1, _win2b, jnp.int32(0))
            cnt = _scan(nwin - 2, 0, cnt, False)
            cnt = _scan(nwin - 1, 1, cnt, True)

            f_t = cnt >> bsh
            rem = cnt & (bw - 1)
            p_t = f_t & 1
            lax.cond(f_t >= 2, _wait_s, lambda _: 0, 0)

            def _drain_full(_):
                _wait_g()
                _fire_s(f_t - 1)
                return 0
            lax.cond(f_t >= 1, _drain_full, lambda _: 0, 0)

            def _drain_part(_):
                _fire_g(f_t)
                _wait_g()

                def _zr(i, _):
                    for q in range(C // 16):
                        rows[i, pl.ds(q * 16, 16)] = zvec
                    return 0
                lax.fori_loop(p_t * bw + rem, p_t * bw + bw, _zr, 0)
                _fire_s(f_t)
                return 0
            lax.cond(rem > 0, _drain_part, lambda _: 0, 0)

            n_out = (jnp.where(f_t > 0, 1, 0).astype(jnp.int32)
                     + jnp.where(rem > 0, 1, 0).astype(jnp.int32))
            lax.fori_loop(0, n_out, _wait_s, 0)

            plsc.subcore_barrier()
            for p in range(ts // 64):
                off = sid * ts + p * 64
                pltpu.sync_copy(acc.at[pl.ds(off, 64)],
                                out.at[pl.ds(lo + off, 64)])

    return k


_segsum_r0 = _make_segsum(NP0, 5120, 2, 491520, 128, 1024)
_segsum_r1 = _make_segsum(NP1, 14336, 12, 884736, 32, 512)


def _pad_edges(src, dst, ep, win):
    e = src.shape[0]
    src = jnp.pad(src, (0, ep - e))
    dst = jnp.pad(dst, (0, ep - e), constant_values=-1)
    return jnp.stack([dst.reshape(-1, win), src.reshape(-1, win)], axis=1)


def kernel(f0, f1, f2, f3, adj0, adj1, adj2, adj3, inc1, inc2, inc3,
           W_same, W_low, W_up, W_lin, b_lin):
    a0 = adj0.astype(jnp.int32)
    a1 = adj1.astype(jnp.int32)
    i1 = inc1.astype(jnp.int32)
    i2 = inc2.astype(jnp.int32)

    bB = 2 * N0
    bC = bB + 2 * N1
    src10 = jnp.concatenate([2 * a0[1], bB + 2 * i1[1] + 1])
    dst10 = jnp.concatenate([a0[0], i1[0]])
    src11 = jnp.concatenate([bB + 2 * a1[1], 2 * i1[0] + 1, bC + i2[1]])
    dst11 = jnp.concatenate([a1[0], i1[1], i2[0]])
    e10 = _pad_edges(src10, dst10, 491520, 1024)
    e11 = _pad_edges(src11, dst11, 884736, 512)

    wA = jnp.concatenate([W_same[0, 0], W_low[0, 0]], axis=1)
    wB = jnp.concatenate([W_same[0, 1], W_up[0, 0]], axis=1)
    tab1 = jnp.concatenate([
        _mm(f0, wA).reshape(-1, C),
        _mm(f1, wB).reshape(-1, C),
        _mm(f2, W_up[0, 1]),
    ], axis=0)

    m0 = _segsum_r0(tab1, e10)
    m1 = _segsum_r1(tab1, e11)

    src20 = jnp.concatenate([a0[1], NP0 + i1[1]])
    dst20 = jnp.concatenate([a0[0], i1[0]])
    e20 = _pad_edges(src20, dst20, 491520, 1024)

    tab2 = jnp.concatenate([
        _mm(m0, W_same[1, 0], sig=True),
        _mm(m1, W_up[1, 0], sig=True),
    ], axis=0)

    m0f = _segsum_r0(tab2, e20)

    return _head(m0f, W_lin, b_lin.astype(jnp.float32), N0)


# speedup vs baseline: 4.3061x; 1.0240x over previous
import functools

import jax
import jax.numpy as jnp
from jax import lax
from jax.experimental import pallas as pl
from jax.experimental.pallas import tpu as pltpu
from jax.experimental.pallas import tpu_sc as plsc

C = 128
N0, N1, N2 = 10000, 160000, 80000
NP0, NP1 = 10240, 186368
NSUB = 16


def _sigmoid(x):
    return 1.0 / (1.0 + jnp.exp(-x))


def _mm_kernel(sig, x_ref, w_ref, o_ref):
    x = x_ref[...]
    if sig:
        x = _sigmoid(x)
    o_ref[...] = jnp.dot(x, w_ref[...], preferred_element_type=jnp.float32)


def _mm(x, w, sig=False, bm=1024):
    n, k = x.shape
    m = w.shape[1]
    return pl.pallas_call(
        functools.partial(_mm_kernel, sig),
        grid=(pl.cdiv(n, bm),),
        in_specs=[pl.BlockSpec((bm, k), lambda i: (i, 0)),
                  pl.BlockSpec((k, m), lambda i: (0, 0))],
        out_specs=pl.BlockSpec((bm, m), lambda i: (i, 0)),
        out_shape=jax.ShapeDtypeStruct((n, m), jnp.float32),
    )(x, w)


def _head_kernel(x_ref, w_ref, b_ref, o_ref):
    x = _sigmoid(x_ref[...])
    logits = jnp.dot(x, w_ref[...], preferred_element_type=jnp.float32)
    logits = logits + b_ref[...]
    z = logits - jnp.max(logits, axis=1, keepdims=True)
    e = jnp.exp(z)
    o_ref[...] = e / jnp.sum(e, axis=1, keepdims=True)


def _head(x, w, b, n_out, bm=1024):
    k = x.shape[1]
    m = w.shape[1]
    return pl.pallas_call(
        _head_kernel,
        grid=(pl.cdiv(n_out, bm),),
        in_specs=[pl.BlockSpec((bm, k), lambda i: (i, 0)),
                  pl.BlockSpec((k, m), lambda i: (0, 0)),
                  pl.BlockSpec((1, m), lambda i: (0, 0))],
        out_specs=pl.BlockSpec((bm, m), lambda i: (i, 0)),
        out_shape=jax.ShapeDtypeStruct((n_out, m), jnp.float32),
    )(x, w, b.reshape(1, m))


def _make_segsum(np_rows, chsize, nch, ep, bw, win):
    assert np_rows == nch * chsize and nch % 2 == 0
    assert chsize % (NSUB * 64) == 0 and ep % (NSUB * win) == 0
    assert bw >= 32 and bw % 16 == 0 and (bw & (bw - 1)) == 0
    bsh = bw.bit_length() - 1
    ts = chsize // NSUB
    share = ep // NSUB
    nwin = share // win
    assert nwin % 2 == 0
    mesh = plsc.VectorSubcoreMesh(core_axis_name="c", subcore_axis_name="s")

    @functools.partial(
        pl.kernel, mesh=mesh,
        out_type=jax.ShapeDtypeStruct((np_rows, C), jnp.float32),
        scratch_types=[
            pltpu.VMEM_SHARED((chsize, C), jnp.float32),
            pltpu.VMEM((2, 2, win), jnp.int32),
            pltpu.VMEM((4, bw), jnp.int32),
            pltpu.VMEM((4, bw), jnp.int32),
            pltpu.VMEM((4 * bw, C), jnp.float32),
            pltpu.VMEM((32, C), jnp.float32),
            pltpu.SemaphoreType.DMA,
            pltpu.SemaphoreType.DMA,
            pltpu.SemaphoreType.DMA,
        ],
        compiler_params=pltpu.CompilerParams(needs_layout_passes=False),
    )
    def k(table, edges, out, acc, ewin, gidx, ldst, rows, zbuf, wsem, gsem,
          ssem):
        cid = lax.axis_index("c")
        sid = lax.axis_index("s")
        zvec = jnp.zeros((16,), jnp.float32)
        zivec = jnp.zeros((16,), jnp.int32)

        def _wait_w():
            pltpu.make_async_copy(edges.at[0], ewin.at[0], wsem).wait()

        def _wait_g():
            pltpu.make_async_copy(table.at[gidx.at[0]],
                                  rows.at[pl.ds(0, bw)], gsem).wait()

        def _wait_s(*_):
            pltpu.make_async_copy(rows.at[pl.ds(0, bw)],
                                  acc.at[ldst.at[0]], ssem).wait()
            return 0

        def _fire_g(f):
            pltpu.async_copy(table.at[gidx.at[f & 3]],
                             rows.at[pl.ds((f & 3) * bw, bw)], gsem)

        def _fire_s(f):
            pltpu.async_copy(rows.at[pl.ds((f & 3) * bw, bw)],
                             acc.at[ldst.at[f & 3]], ssem, add=True)

        def _zb(i, _):
            zbuf[i // 8, pl.ds((i % 8) * 16, 16)] = zvec
            return 0
        lax.fori_loop(0, 256, _zb, 0)
        for j in range(4):
            for q in range(bw // 16):
                gidx[j, pl.ds(q * 16, 16)] = zivec
                ldst[j, pl.ds(q * 16, 16)] = zivec

        for kk in range(nch // 2):
            lo = (2 * kk + cid) * chsize
            hi = lo + chsize
            for p in range(ts // 32):
                pltpu.sync_copy(zbuf, acc.at[pl.ds(sid * ts + p * 32, 32)])
            plsc.subcore_barrier()

            wbase = sid * nwin
            pltpu.async_copy(edges.at[wbase], ewin.at[0], wsem)

            def _fire(c):
                f = (c >> bsh) - 1
                lax.cond(f >= 3, _wait_s, lambda _: 0, 0)
                _fire_g(f)

                def _gs(_):
                    _wait_g()
                    _fire_s(f - 1)
                    return 0
                lax.cond(f >= 1, _gs, lambda _: 0, 0)
                return c

            def _scan(w, par, cnt, last):
                _wait_w()
                if not last:
                    pltpu.async_copy(edges.at[wbase + w + 1],
                                     ewin.at[1 - par], wsem)

                def _vec(i, cnt):
                    d = ewin[par, 0, pl.ds(i * 16, 16)]
                    s = ewin[par, 1, pl.ds(i * 16, 16)]
                    m = (d >= lo) & (d < hi)
                    mi = jnp.where(m, jnp.full((16,), 1, jnp.int32),
                                   jnp.full((16,), 0, jnp.int32))
                    pos = cnt + plsc.cumsum(mi) - 1
                    pj = (pos >> bsh) & 3
                    pq = pos & (bw - 1)
                    plsc.store_scatter(gidx, [pj, pq], s, mask=m)
                    plsc.store_scatter(ldst, [pj, pq], d - lo, mask=m)
                    newc = cnt + jnp.sum(mi)
                    return lax.cond((newc >> bsh) > (cnt >> bsh), _fire,
                                    lambda c: c, newc)
                return lax.fori_loop(0, win // 16, _vec, cnt)

            def _win2b(w2, cnt):
                w = 2 * w2
                cnt = _scan(w, 0, cnt, False)
                return _scan(w + 1, 1, cnt, False)

            cnt = lax.fori_loop(0, nwin // 2 - 1, _win2b, jnp.int32(0))
            cnt = _scan(nwin - 2, 0, cnt, False)
            cnt = _scan(nwin - 1, 1, cnt, True)

            f_t = cnt >> bsh
            rem = cnt & (bw - 1)
            lax.cond(f_t >= 3, _wait_s, lambda _: 0, 0)

            def _drain_full(_):
                _wait_g()
                _fire_s(f_t - 1)
                return 0
            lax.cond(f_t >= 1, _drain_full, lambda _: 0, 0)

            def _drain_part(_):
                _fire_g(f_t)
                _wait_g()
                base = (f_t & 3) * bw

                def _zr(i, _):
                    for q in range(C // 16):
                        rows[i, pl.ds(q * 16, 16)] = zvec
                    return 0
                lax.fori_loop(base + rem, base + bw, _zr, 0)
                _fire_s(f_t)
                return 0
            lax.cond(rem > 0, _drain_part, lambda _: 0, 0)

            n_out = (jnp.minimum(f_t, 2)
                     + jnp.where(rem > 0, 1, 0).astype(jnp.int32))
            lax.fori_loop(0, n_out, _wait_s, 0)

            plsc.subcore_barrier()
            for p in range(ts // 64):
                off = sid * ts + p * 64
                pltpu.sync_copy(acc.at[pl.ds(off, 64)],
                                out.at[pl.ds(lo + off, 64)])

    return k


_segsum_r0 = _make_segsum(NP0, 5120, 2, 491520, 128, 1024)
_segsum_r1 = _make_segsum(NP1, 13312, 14, 884736, 32, 512)


def _pad_edges(src, dst, ep, win):
    e = src.shape[0]
    src = jnp.pad(src, (0, ep - e))
    dst = jnp.pad(dst, (0, ep - e), constant_values=-1)
    return jnp.stack([dst.reshape(-1, win), src.reshape(-1, win)], axis=1)


def kernel(f0, f1, f2, f3, adj0, adj1, adj2, adj3, inc1, inc2, inc3,
           W_same, W_low, W_up, W_lin, b_lin):
    a0 = adj0.astype(jnp.int32)
    a1 = adj1.astype(jnp.int32)
    i1 = inc1.astype(jnp.int32)
    i2 = inc2.astype(jnp.int32)

    bB = 2 * N0
    bC = bB + 2 * N1
    src10 = jnp.concatenate([2 * a0[1], bB + 2 * i1[1] + 1])
    dst10 = jnp.concatenate([a0[0], i1[0]])
    src11 = jnp.concatenate([bB + 2 * a1[1], 2 * i1[0] + 1, bC + i2[1]])
    dst11 = jnp.concatenate([a1[0], i1[1], i2[0]])
    e10 = _pad_edges(src10, dst10, 491520, 1024)
    e11 = _pad_edges(src11, dst11, 884736, 512)

    wA = jnp.concatenate([W_same[0, 0], W_low[0, 0]], axis=1)
    wB = jnp.concatenate([W_same[0, 1], W_up[0, 0]], axis=1)
    tab1 = jnp.concatenate([
        _mm(f0, wA).reshape(-1, C),
        _mm(f1, wB).reshape(-1, C),
        _mm(f2, W_up[0, 1]),
    ], axis=0)

    m0 = _segsum_r0(tab1, e10)
    m1 = _segsum_r1(tab1, e11)

    src20 = jnp.concatenate([a0[1], NP0 + i1[1]])
    dst20 = jnp.concatenate([a0[0], i1[0]])
    e20 = _pad_edges(src20, dst20, 491520, 1024)

    tab2 = jnp.concatenate([
        _mm(m0, W_same[1, 0], sig=True),
        _mm(m1, W_up[1, 0], sig=True),
    ], axis=0)

    m0f = _segsum_r0(tab2, e20)

    return _head(m0f, W_lin, b_lin.astype(jnp.float32), N0)


# speedup vs baseline: 4.7346x; 1.0995x over previous
import functools

import jax
import jax.numpy as jnp
from jax import lax
from jax.experimental import pallas as pl
from jax.experimental.pallas import tpu as pltpu
from jax.experimental.pallas import tpu_sc as plsc

C = 128
N0, N1, N2 = 10000, 160000, 80000
NP0, NP1 = 10240, 186368
NSUB = 16


def _sigmoid(x):
    return 1.0 / (1.0 + jnp.exp(-x))


def _tab1_kernel(f0_ref, f1_ref, x2_ref, wa_ref, wb_ref, w2_ref, o_ref):
    i = pl.program_id(0)

    @pl.when(i < 5)
    def _a():
        o_ref[...] = jnp.dot(f0_ref[...], wa_ref[...],
                             preferred_element_type=jnp.float32)

    @pl.when((i >= 5) & (i < 85))
    def _b():
        o_ref[...] = jnp.dot(f1_ref[...], wb_ref[...],
                             preferred_element_type=jnp.float32)

    @pl.when(i >= 85)
    def _c():
        o_ref[...] = jnp.dot(x2_ref[...], w2_ref[...],
                             preferred_element_type=jnp.float32)


def _build_tab1(f0, f1, f2, wa, wb, wu):
    x2 = f2.reshape(40000, 256)
    z = jnp.zeros((C, C), jnp.float32)
    w2 = jnp.block([[wu, z], [z, wu]])
    t = pl.pallas_call(
        _tab1_kernel,
        grid=(105,),
        in_specs=[
            pl.BlockSpec((2000, 128), lambda i: (jnp.minimum(i, 4), 0)),
            pl.BlockSpec((2000, 128),
                         lambda i: (jnp.clip(i - 5, 0, 79), 0)),
            pl.BlockSpec((2000, 256),
                         lambda i: (jnp.clip(i - 85, 0, 19), 0)),
            pl.BlockSpec((128, 256), lambda i: (0, 0)),
            pl.BlockSpec((128, 256), lambda i: (0, 0)),
            pl.BlockSpec((256, 256), lambda i: (0, 0)),
        ],
        out_specs=pl.BlockSpec((2000, 256), lambda i: (i, 0)),
        out_shape=jax.ShapeDtypeStruct((210000, 256), jnp.float32),
    )(f0, f1, x2, wa, wb, w2)
    return t.reshape(420000, C)


def _tab2_kernel(m0_ref, m1_ref, w0_ref, w1_ref, o_ref):
    i = pl.program_id(0)

    @pl.when(i < 5)
    def _a():
        o_ref[...] = jnp.dot(_sigmoid(m0_ref[...]), w0_ref[...],
                             preferred_element_type=jnp.float32)

    @pl.when(i >= 5)
    def _b():
        o_ref[...] = jnp.dot(_sigmoid(m1_ref[...]), w1_ref[...],
                             preferred_element_type=jnp.float32)


def _build_tab2(m0, m1, w0, w1):
    return pl.pallas_call(
        _tab2_kernel,
        grid=(84,),
        in_specs=[
            pl.BlockSpec((2048, 128), lambda i: (jnp.minimum(i, 4), 0)),
            pl.BlockSpec((2048, 128),
                         lambda i: (jnp.clip(i - 5, 0, 78), 0)),
            pl.BlockSpec((128, 128), lambda i: (0, 0)),
            pl.BlockSpec((128, 128), lambda i: (0, 0)),
        ],
        out_specs=pl.BlockSpec((2048, 128), lambda i: (i, 0)),
        out_shape=jax.ShapeDtypeStruct((172032, 128), jnp.float32),
    )(m0, m1, w0, w1)


def _head_kernel(x_ref, w_ref, b_ref, o_ref):
    x = _sigmoid(x_ref[...])
    logits = jnp.dot(x, w_ref[...], preferred_element_type=jnp.float32)
    logits = logits + b_ref[...]
    z = logits - jnp.max(logits, axis=1, keepdims=True)
    e = jnp.exp(z)
    o_ref[...] = e / jnp.sum(e, axis=1, keepdims=True)


def _head(x, w, b, n_out, bm=1024):
    k = x.shape[1]
    m = w.shape[1]
    return pl.pallas_call(
        _head_kernel,
        grid=(pl.cdiv(n_out, bm),),
        in_specs=[pl.BlockSpec((bm, k), lambda i: (i, 0)),
                  pl.BlockSpec((k, m), lambda i: (0, 0)),
                  pl.BlockSpec((1, m), lambda i: (0, 0))],
        out_specs=pl.BlockSpec((bm, m), lambda i: (i, 0)),
        out_shape=jax.ShapeDtypeStruct((n_out, m), jnp.float32),
    )(x, w, b.reshape(1, m))


def _make_segsum(np_rows, chsize, nch, ep, bw, win):
    assert np_rows == nch * chsize and nch % 2 == 0
    assert chsize % (NSUB * 64) == 0 and ep % (NSUB * win) == 0
    assert bw >= 32 and bw % 16 == 0 and (bw & (bw - 1)) == 0
    bsh = bw.bit_length() - 1
    ts = chsize // NSUB
    share = ep // NSUB
    nwin = share // win
    assert nwin % 2 == 0
    mesh = plsc.VectorSubcoreMesh(core_axis_name="c", subcore_axis_name="s")

    @functools.partial(
        pl.kernel, mesh=mesh,
        out_type=jax.ShapeDtypeStruct((np_rows, C), jnp.float32),
        scratch_types=[
            pltpu.VMEM_SHARED((chsize, C), jnp.float32),
            pltpu.VMEM((2, 2, win), jnp.int32),
            pltpu.VMEM((4, bw), jnp.int32),
            pltpu.VMEM((4, bw), jnp.int32),
            pltpu.VMEM((4 * bw, C), jnp.float32),
            pltpu.VMEM((32, C), jnp.float32),
            pltpu.SemaphoreType.DMA,
            pltpu.SemaphoreType.DMA,
            pltpu.SemaphoreType.DMA,
        ],
        compiler_params=pltpu.CompilerParams(needs_layout_passes=False),
    )
    def k(table, edges, out, acc, ewin, gidx, ldst, rows, zbuf, wsem, gsem,
          ssem):
        cid = lax.axis_index("c")
        sid = lax.axis_index("s")
        zvec = jnp.zeros((16,), jnp.float32)
        zivec = jnp.zeros((16,), jnp.int32)

        def _wait_w():
            pltpu.make_async_copy(edges.at[0], ewin.at[0], wsem).wait()

        def _wait_g():
            pltpu.make_async_copy(table.at[gidx.at[0]],
                                  rows.at[pl.ds(0, bw)], gsem).wait()

        def _wait_s(*_):
            pltpu.make_async_copy(rows.at[pl.ds(0, bw)],
                                  acc.at[ldst.at[0]], ssem).wait()
            return 0

        def _fire_g(f):
            pltpu.async_copy(table.at[gidx.at[f & 3]],
                             rows.at[pl.ds((f & 3) * bw, bw)], gsem)

        def _fire_s(f):
            pltpu.async_copy(rows.at[pl.ds((f & 3) * bw, bw)],
                             acc.at[ldst.at[f & 3]], ssem, add=True)

        def _zb(i, _):
            zbuf[i // 8, pl.ds((i % 8) * 16, 16)] = zvec
            return 0
        lax.fori_loop(0, 256, _zb, 0)
        for j in range(4):
            for q in range(bw // 16):
                gidx[j, pl.ds(q * 16, 16)] = zivec
                ldst[j, pl.ds(q * 16, 16)] = zivec

        for kk in range(nch // 2):
            lo = (2 * kk + cid) * chsize
            hi = lo + chsize
            for p in range(ts // 32):
                pltpu.sync_copy(zbuf, acc.at[pl.ds(sid * ts + p * 32, 32)])
            plsc.subcore_barrier()

            wbase = sid * nwin
            pltpu.async_copy(edges.at[wbase], ewin.at[0], wsem)

            def _fire(c):
                f = (c >> bsh) - 1
                lax.cond(f >= 3, _wait_s, lambda _: 0, 0)
                _fire_g(f)

                def _gs(_):
                    _wait_g()
                    _fire_s(f - 1)
                    return 0
                lax.cond(f >= 1, _gs, lambda _: 0, 0)
                return c

            def _scan(w, par, cnt, last):
                _wait_w()
                if not last:
                    pltpu.async_copy(edges.at[wbase + w + 1],
                                     ewin.at[1 - par], wsem)

                def _vec(i, cnt):
                    d = ewin[par, 0, pl.ds(i * 16, 16)]
                    s = ewin[par, 1, pl.ds(i * 16, 16)]
                    m = (d >= lo) & (d < hi)
                    mi = jnp.where(m, jnp.full((16,), 1, jnp.int32),
                                   jnp.full((16,), 0, jnp.int32))
                    pos = cnt + plsc.cumsum(mi) - 1
                    pj = (pos >> bsh) & 3
                    pq = pos & (bw - 1)
                    plsc.store_scatter(gidx, [pj, pq], s, mask=m)
                    plsc.store_scatter(ldst, [pj, pq], d - lo, mask=m)
                    newc = cnt + jnp.sum(mi)
                    return lax.cond((newc >> bsh) > (cnt >> bsh), _fire,
                                    lambda c: c, newc)
                return lax.fori_loop(0, win // 16, _vec, cnt)

            def _win2b(w2, cnt):
                w = 2 * w2
                cnt = _scan(w, 0, cnt, False)
                return _scan(w + 1, 1, cnt, False)

            cnt = lax.fori_loop(0, nwin // 2 - 1, _win2b, jnp.int32(0))
            cnt = _scan(nwin - 2, 0, cnt, False)
            cnt = _scan(nwin - 1, 1, cnt, True)

            f_t = cnt >> bsh
            rem = cnt & (bw - 1)
            lax.cond(f_t >= 3, _wait_s, lambda _: 0, 0)

            def _drain_full(_):
                _wait_g()
                _fire_s(f_t - 1)
                return 0
            lax.cond(f_t >= 1, _drain_full, lambda _: 0, 0)

            def _drain_part(_):
                _fire_g(f_t)
                _wait_g()
                base = (f_t & 3) * bw

                def _zr(i, _):
                    for q in range(C // 16):
                        rows[i, pl.ds(q * 16, 16)] = zvec
                    return 0
                lax.fori_loop(base + rem, base + bw, _zr, 0)
                _fire_s(f_t)
                return 0
            lax.cond(rem > 0, _drain_part, lambda _: 0, 0)

            n_out = (jnp.minimum(f_t, 2)
                     + jnp.where(rem > 0, 1, 0).astype(jnp.int32))
            lax.fori_loop(0, n_out, _wait_s, 0)

            plsc.subcore_barrier()
            for p in range(ts // 64):
                off = sid * ts + p * 64
                pltpu.sync_copy(acc.at[pl.ds(off, 64)],
                                out.at[pl.ds(lo + off, 64)])

    return k


_segsum_r0 = _make_segsum(NP0, 5120, 2, 491520, 128, 1024)
_segsum_r1 = _make_segsum(NP1, 13312, 14, 884736, 32, 512)


def _pad_edges(src, dst, ep, win):
    e = src.shape[0]
    src = jnp.pad(src, (0, ep - e))
    dst = jnp.pad(dst, (0, ep - e), constant_values=-1)
    return jnp.stack([dst.reshape(-1, win), src.reshape(-1, win)], axis=1)


def kernel(f0, f1, f2, f3, adj0, adj1, adj2, adj3, inc1, inc2, inc3,
           W_same, W_low, W_up, W_lin, b_lin):
    a0 = adj0.astype(jnp.int32)
    a1 = adj1.astype(jnp.int32)
    i1 = inc1.astype(jnp.int32)
    i2 = inc2.astype(jnp.int32)

    bB = 2 * N0
    bC = bB + 2 * N1
    src10 = jnp.concatenate([2 * a0[1], bB + 2 * i1[1] + 1])
    dst10 = jnp.concatenate([a0[0], i1[0]])
    src11 = jnp.concatenate([bB + 2 * a1[1], 2 * i1[0] + 1, bC + i2[1]])
    dst11 = jnp.concatenate([a1[0], i1[1], i2[0]])
    e10 = _pad_edges(src10, dst10, 491520, 1024)
    e11 = _pad_edges(src11, dst11, 884736, 512)

    wA = jnp.concatenate([W_same[0, 0], W_low[0, 0]], axis=1)
    wB = jnp.concatenate([W_same[0, 1], W_up[0, 0]], axis=1)
    tab1 = _build_tab1(f0, f1, f2, wA, wB, W_up[0, 1])

    m0 = _segsum_r0(tab1, e10)
    m1 = _segsum_r1(tab1, e11)

    src20 = jnp.concatenate([a0[1], NP0 + i1[1]])
    dst20 = jnp.concatenate([a0[0], i1[0]])
    e20 = _pad_edges(src20, dst20, 491520, 1024)

    tab2 = _build_tab2(m0, m1, W_same[1, 0], W_up[1, 0])

    m0f = _segsum_r0(tab2, e20)

    return _head(m0f, W_lin, b_lin.astype(jnp.float32), N0)


# speedup vs baseline: 4.7944x; 1.0126x over previous
import functools

import jax
import jax.numpy as jnp
from jax import lax
from jax.experimental import pallas as pl
from jax.experimental.pallas import tpu as pltpu
from jax.experimental.pallas import tpu_sc as plsc

C = 128
N0, N1, N2 = 10000, 160000, 80000
NP0, NP1 = 10240, 186368
NSUB = 16


def _sigmoid(x):
    return 1.0 / (1.0 + jnp.exp(-x))


def _tab1_kernel(f0_ref, f1_ref, x2_ref, wa_ref, wb_ref, w2_ref, o_ref):
    i = pl.program_id(0)

    @pl.when(i < 5)
    def _a():
        o_ref[...] = jnp.dot(f0_ref[...], wa_ref[...],
                             preferred_element_type=jnp.float32)

    @pl.when((i >= 5) & (i < 85))
    def _b():
        o_ref[...] = jnp.dot(f1_ref[...], wb_ref[...],
                             preferred_element_type=jnp.float32)

    @pl.when(i >= 85)
    def _c():
        o_ref[...] = jnp.dot(x2_ref[...], w2_ref[...],
                             preferred_element_type=jnp.float32)


def _build_tab1(f0, f1, f2, wa, wb, wu):
    x2 = f2.reshape(40000, 256)
    z = jnp.zeros((C, C), jnp.float32)
    w2 = jnp.block([[wu, z], [z, wu]])
    t = pl.pallas_call(
        _tab1_kernel,
        grid=(105,),
        in_specs=[
            pl.BlockSpec((2000, 128), lambda i: (jnp.minimum(i, 4), 0)),
            pl.BlockSpec((2000, 128),
                         lambda i: (jnp.clip(i - 5, 0, 79), 0)),
            pl.BlockSpec((2000, 256),
                         lambda i: (jnp.clip(i - 85, 0, 19), 0)),
            pl.BlockSpec((128, 256), lambda i: (0, 0)),
            pl.BlockSpec((128, 256), lambda i: (0, 0)),
            pl.BlockSpec((256, 256), lambda i: (0, 0)),
        ],
        out_specs=pl.BlockSpec((2000, 256), lambda i: (i, 0)),
        out_shape=jax.ShapeDtypeStruct((210000, 256), jnp.float32),
    )(f0, f1, x2, wa, wb, w2)
    return t.reshape(420000, C)


def _tab2_kernel(m0_ref, m1_ref, w0_ref, w1_ref, o_ref):
    i = pl.program_id(0)

    @pl.when(i < 5)
    def _a():
        o_ref[...] = jnp.dot(_sigmoid(m0_ref[...]), w0_ref[...],
                             preferred_element_type=jnp.float32)

    @pl.when(i >= 5)
    def _b():
        o_ref[...] = jnp.dot(_sigmoid(m1_ref[...]), w1_ref[...],
                             preferred_element_type=jnp.float32)


def _build_tab2(m0, m1, w0, w1):
    return pl.pallas_call(
        _tab2_kernel,
        grid=(84,),
        in_specs=[
            pl.BlockSpec((2048, 128), lambda i: (jnp.minimum(i, 4), 0)),
            pl.BlockSpec((2048, 128),
                         lambda i: (jnp.clip(i - 5, 0, 78), 0)),
            pl.BlockSpec((128, 128), lambda i: (0, 0)),
            pl.BlockSpec((128, 128), lambda i: (0, 0)),
        ],
        out_specs=pl.BlockSpec((2048, 128), lambda i: (i, 0)),
        out_shape=jax.ShapeDtypeStruct((172032, 128), jnp.float32),
    )(m0, m1, w0, w1)


def _head_kernel(x_ref, w_ref, b_ref, o_ref):
    x = _sigmoid(x_ref[...])
    logits = jnp.dot(x, w_ref[...], preferred_element_type=jnp.float32)
    logits = logits + b_ref[...]
    z = logits - jnp.max(logits, axis=1, keepdims=True)
    e = jnp.exp(z)
    o_ref[...] = e / jnp.sum(e, axis=1, keepdims=True)


def _head(x, w, b, n_out, bm=1024):
    k = x.shape[1]
    m = w.shape[1]
    return pl.pallas_call(
        _head_kernel,
        grid=(pl.cdiv(n_out, bm),),
        in_specs=[pl.BlockSpec((bm, k), lambda i: (i, 0)),
                  pl.BlockSpec((k, m), lambda i: (0, 0)),
                  pl.BlockSpec((1, m), lambda i: (0, 0))],
        out_specs=pl.BlockSpec((bm, m), lambda i: (i, 0)),
        out_shape=jax.ShapeDtypeStruct((n_out, m), jnp.float32),
    )(x, w, b.reshape(1, m))


def _make_segsum(np_rows, chsize, nch, ep, bw, win):
    assert np_rows == nch * chsize and nch % 2 == 0
    assert chsize % (NSUB * 64) == 0 and ep % (NSUB * win) == 0
    assert bw >= 32 and bw % 16 == 0 and (bw & (bw - 1)) == 0
    bsh = bw.bit_length() - 1
    ts = chsize // NSUB
    share = ep // NSUB
    nwin = share // win
    assert nwin % 2 == 0
    mesh = plsc.VectorSubcoreMesh(core_axis_name="c", subcore_axis_name="s")

    @functools.partial(
        pl.kernel, mesh=mesh,
        out_type=jax.ShapeDtypeStruct((np_rows, C), jnp.float32),
        scratch_types=[
            pltpu.VMEM_SHARED((chsize, C), jnp.float32),
            pltpu.VMEM((2, 2, win), jnp.int32),
            pltpu.VMEM((4, bw), jnp.int32),
            pltpu.VMEM((4, bw), jnp.int32),
            pltpu.VMEM((4 * bw, C), jnp.float32),
            pltpu.VMEM((32, C), jnp.float32),
            pltpu.SemaphoreType.DMA,
            pltpu.SemaphoreType.DMA,
            pltpu.SemaphoreType.DMA,
        ],
        compiler_params=pltpu.CompilerParams(needs_layout_passes=False),
    )
    def k(table, edges, out, acc, ewin, gidx, ldst, rows, zbuf, wsem, gsem,
          ssem):
        cid = lax.axis_index("c")
        sid = lax.axis_index("s")
        zvec = jnp.zeros((16,), jnp.float32)
        zivec = jnp.zeros((16,), jnp.int32)

        def _wait_w():
            pltpu.make_async_copy(edges.at[0], ewin.at[0], wsem).wait()

        def _wait_g():
            pltpu.make_async_copy(table.at[gidx.at[0]],
                                  rows.at[pl.ds(0, bw)], gsem).wait()

        def _wait_s(*_):
            pltpu.make_async_copy(rows.at[pl.ds(0, bw)],
                                  acc.at[ldst.at[0]], ssem).wait()
            return 0

        def _fire_g(f):
            pltpu.async_copy(table.at[gidx.at[f & 3]],
                             rows.at[pl.ds((f & 3) * bw, bw)], gsem)

        def _fire_s(f):
            pltpu.async_copy(rows.at[pl.ds((f & 3) * bw, bw)],
                             acc.at[ldst.at[f & 3]], ssem, add=True)

        def _zb(i, _):
            zbuf[i // 8, pl.ds((i % 8) * 16, 16)] = zvec
            return 0
        lax.fori_loop(0, 256, _zb, 0)
        for j in range(4):
            for q in range(bw // 16):
                gidx[j, pl.ds(q * 16, 16)] = zivec
                ldst[j, pl.ds(q * 16, 16)] = zivec

        for kk in range(nch // 2):
            lo = (2 * kk + cid) * chsize
            hi = lo + chsize
            for p in range(ts // 32):
                pltpu.async_copy(zbuf, acc.at[pl.ds(sid * ts + p * 32, 32)],
                                 wsem)
            for p in range(ts // 32):
                pltpu.make_async_copy(zbuf, acc.at[pl.ds(0, 32)],
                                      wsem).wait()
            plsc.subcore_barrier()

            wbase = sid * nwin
            pltpu.async_copy(edges.at[wbase], ewin.at[0], wsem)

            def _fire(c):
                f = (c >> bsh) - 1
                lax.cond(f >= 3, _wait_s, lambda _: 0, 0)
                _fire_g(f)

                def _gs(_):
                    _wait_g()
                    _fire_s(f - 1)
                    return 0
                lax.cond(f >= 1, _gs, lambda _: 0, 0)
                return c

            def _scan(w, par, cnt, last):
                _wait_w()
                if not last:
                    pltpu.async_copy(edges.at[wbase + w + 1],
                                     ewin.at[1 - par], wsem)

                def _vec(i, cnt):
                    d = ewin[par, 0, pl.ds(i * 16, 16)]
                    s = ewin[par, 1, pl.ds(i * 16, 16)]
                    m = (d >= lo) & (d < hi)
                    mi = jnp.where(m, jnp.full((16,), 1, jnp.int32),
                                   jnp.full((16,), 0, jnp.int32))
                    pos = cnt + plsc.cumsum(mi) - 1
                    pj = (pos >> bsh) & 3
                    pq = pos & (bw - 1)
                    plsc.store_scatter(gidx, [pj, pq], s, mask=m)
                    plsc.store_scatter(ldst, [pj, pq], d - lo, mask=m)
                    newc = cnt + jnp.sum(mi)
                    return lax.cond((newc >> bsh) > (cnt >> bsh), _fire,
                                    lambda c: c, newc)
                return lax.fori_loop(0, win // 16, _vec, cnt)

            def _win2b(w2, cnt):
                w = 2 * w2
                cnt = _scan(w, 0, cnt, False)
                return _scan(w + 1, 1, cnt, False)

            cnt = lax.fori_loop(0, nwin // 2 - 1, _win2b, jnp.int32(0))
            cnt = _scan(nwin - 2, 0, cnt, False)
            cnt = _scan(nwin - 1, 1, cnt, True)

            f_t = cnt >> bsh
            rem = cnt & (bw - 1)
            lax.cond(f_t >= 3, _wait_s, lambda _: 0, 0)

            def _drain_full(_):
                _wait_g()
                _fire_s(f_t - 1)
                return 0
            lax.cond(f_t >= 1, _drain_full, lambda _: 0, 0)

            def _drain_part(_):
                _fire_g(f_t)
                _wait_g()
                base = (f_t & 3) * bw

                def _zr(i, _):
                    for q in range(C // 16):
                        rows[i, pl.ds(q * 16, 16)] = zvec
                    return 0
                lax.fori_loop(base + rem, base + bw, _zr, 0)
                _fire_s(f_t)
                return 0
            lax.cond(rem > 0, _drain_part, lambda _: 0, 0)

            n_out = (jnp.minimum(f_t, 2)
                     + jnp.where(rem > 0, 1, 0).astype(jnp.int32))
            lax.fori_loop(0, n_out, _wait_s, 0)

            plsc.subcore_barrier()
            for p in range(ts // 64):
                off = sid * ts + p * 64
                pltpu.async_copy(acc.at[pl.ds(off, 64)],
                                 out.at[pl.ds(lo + off, 64)], wsem)
            for p in range(ts // 64):
                pltpu.make_async_copy(acc.at[pl.ds(0, 64)],
                                      out.at[pl.ds(0, 64)], wsem).wait()

    return k


_segsum_r0 = _make_segsum(NP0, 5120, 2, 491520, 128, 1024)
_segsum_r1 = _make_segsum(NP1, 13312, 14, 884736, 32, 512)


def _pad_edges(src, dst, ep, win):
    e = src.shape[0]
    src = jnp.pad(src, (0, ep - e))
    dst = jnp.pad(dst, (0, ep - e), constant_values=-1)
    return jnp.stack([dst.reshape(-1, win), src.reshape(-1, win)], axis=1)


def kernel(f0, f1, f2, f3, adj0, adj1, adj2, adj3, inc1, inc2, inc3,
           W_same, W_low, W_up, W_lin, b_lin):
    a0 = adj0.astype(jnp.int32)
    a1 = adj1.astype(jnp.int32)
    i1 = inc1.astype(jnp.int32)
    i2 = inc2.astype(jnp.int32)

    bB = 2 * N0
    bC = bB + 2 * N1
    src10 = jnp.concatenate([2 * a0[1], bB + 2 * i1[1] + 1])
    dst10 = jnp.concatenate([a0[0], i1[0]])
    src11 = jnp.concatenate([bB + 2 * a1[1], 2 * i1[0] + 1, bC + i2[1]])
    dst11 = jnp.concatenate([a1[0], i1[1], i2[0]])
    e10 = _pad_edges(src10, dst10, 491520, 1024)
    e11 = _pad_edges(src11, dst11, 884736, 512)

    wA = jnp.concatenate([W_same[0, 0], W_low[0, 0]], axis=1)
    wB = jnp.concatenate([W_same[0, 1], W_up[0, 0]], axis=1)
    tab1 = _build_tab1(f0, f1, f2, wA, wB, W_up[0, 1])

    m0 = _segsum_r0(tab1, e10)
    m1 = _segsum_r1(tab1, e11)

    src20 = jnp.concatenate([a0[1], NP0 + i1[1]])
    dst20 = jnp.concatenate([a0[0], i1[0]])
    e20 = _pad_edges(src20, dst20, 491520, 1024)

    tab2 = _build_tab2(m0, m1, W_same[1, 0], W_up[1, 0])

    m0f = _segsum_r0(tab2, e20)

    return _head(m0f, W_lin, b_lin.astype(jnp.float32), N0)
